# Initial kernel scaffold; baseline (speedup 1.0000x reference)
import jax
import jax.numpy as jnp
from jax.experimental import pallas as pl


def kernel(vertices, opt_d_rotations, opt_d_translations, weights, nodes_idx, influence_nodes_idx, one_ring_neigh):
    raise NotImplementedError("write your pallas kernel here")


# speedup vs baseline: 5.8173x; 5.8173x over previous
import functools

import jax
import jax.numpy as jnp
from jax import lax
from jax.experimental import pallas as pl
from jax.experimental.pallas import tpu as pltpu
from jax.experimental.pallas import tpu_sc as plsc

NC = 2
NS = 16
NW = NC * NS
L = 16
SL = 128
PW = 128

N = 50000
M = 25000
K = 3
NEIGH = 9

BN = 1664
N_PAD = NW * BN
N_V = 51200
BMW = 896
M_PAD = NW * BMW
SUB = 1024
SCH = 7
BA = SUB * SCH
MN_PAD = NW * BA
MROWS = 128

_mesh = plsc.VectorSubcoreMesh(core_axis_name="c", subcore_axis_name="s")
_params = pltpu.CompilerParams(needs_layout_passes=False)


def _col(c):
    return jnp.full((L,), c, jnp.int32)


@functools.partial(
    pl.kernel,
    out_type=jax.ShapeDtypeStruct((M_PAD, PW), jnp.float32),
    mesh=_mesh,
    compiler_params=_params,
    scratch_types=[
        pltpu.VMEM((N_V,), jnp.float32),
        pltpu.VMEM((BMW,), jnp.int32),
        pltpu.VMEM((BMW,), jnp.float32),
        pltpu.VMEM((BMW,), jnp.float32),
        pltpu.VMEM((BMW,), jnp.float32),
        pltpu.VMEM((BMW * 9,), jnp.float32),
        pltpu.VMEM((BMW * 3,), jnp.float32),
        pltpu.VMEM((SL, PW), jnp.float32),
    ],
)
def _build_table(nidx, rflat, tflat, vt, p_out,
                 vfull_v, nidx_v, nx_v, ny_v, nz_v, rr_v, tr_v, p_v):
    sid = lax.axis_index("s")
    cid = lax.axis_index("c")
    wid = sid * NC + cid
    lanes = lax.iota(jnp.int32, L)
    mb = wid * BMW

    pltpu.sync_copy(nidx.at[pl.ds(mb, BMW)], nidx_v)
    for c, nc_v in enumerate((nx_v, ny_v, nz_v)):
        pltpu.sync_copy(vt.at[pl.ds(c * N_PAD, N_V)], vfull_v)

        def gbody(g, carry, nc_v=nc_v):
            iv = nidx_v[pl.ds(g * L, L)]
            nc_v[pl.ds(g * L, L)] = plsc.load_gather(vfull_v, [iv])
            return carry

        lax.fori_loop(0, BMW // L, gbody, 0)
    pltpu.sync_copy(rflat.at[pl.ds(mb * 9, BMW * 9)], rr_v)
    pltpu.sync_copy(tflat.at[pl.ds(mb * 3, BMW * 3)], tr_v)

    def pslice(j, carry):
        def pbody(g, c2):
            row = g * L + lanes
            b = j * SL + g * L
            grow = b + lanes
            plsc.store_scatter(p_v, [row, _col(0)], nx_v[pl.ds(b, L)])
            plsc.store_scatter(p_v, [row, _col(1)], ny_v[pl.ds(b, L)])
            plsc.store_scatter(p_v, [row, _col(2)], nz_v[pl.ds(b, L)])
            f3 = grow * 3
            f9 = grow * 9
            for c in range(3):
                plsc.store_scatter(p_v, [row, _col(3 + c)],
                                   plsc.load_gather(tr_v, [f3 + c]))
            for c in range(9):
                plsc.store_scatter(p_v, [row, _col(6 + c)],
                                   plsc.load_gather(rr_v, [f9 + c]))
            return c2

        lax.fori_loop(0, SL // L, pbody, 0)
        pltpu.sync_copy(p_v, p_out.at[pl.ds(mb + j * SL, SL)])
        return carry

    lax.fori_loop(0, BMW // SL, pslice, 0)


@functools.partial(
    pl.kernel,
    out_type=(
        jax.ShapeDtypeStruct((N_PAD * 3,), jnp.float32),
        jax.ShapeDtypeStruct((NW * 2 * L,), jnp.float32),
    ),
    mesh=_mesh,
    compiler_params=_params,
    scratch_types=[
        pltpu.VMEM((SL,), jnp.int32),
        pltpu.VMEM((BN,), jnp.float32),
        pltpu.VMEM((BN,), jnp.float32),
        pltpu.VMEM((BN,), jnp.float32),
        pltpu.VMEM((BN,), jnp.float32),
        pltpu.VMEM((SL, PW), jnp.float32),
        pltpu.VMEM((MROWS, PW), jnp.float32),
        pltpu.VMEM((BN * 3,), jnp.float32),
        pltpu.VMEM((SUB,), jnp.int32),
        pltpu.VMEM((2 * L,), jnp.float32),
        pltpu.SemaphoreType.DMA,
    ],
)
def _warp_losses(p_tab, vt, inft, wt, ornf, mrep, out_w, out_loss,
                 gidx_v, vx_v, vy_v, vz_v, w_v, rows_v, mrows_v, out_v,
                 mi_v, loss_v, sem):
    sid = lax.axis_index("s")
    cid = lax.axis_index("c")
    wid = sid * NC + cid
    lanes = lax.iota(jnp.int32, L)
    lanes3 = lanes * 3

    base = wid * BN
    pltpu.sync_copy(vt.at[pl.ds(base, BN)], vx_v)
    pltpu.sync_copy(vt.at[pl.ds(N_PAD + base, BN)], vy_v)
    pltpu.sync_copy(vt.at[pl.ds(2 * N_PAD + base, BN)], vz_v)
    for k in range(K):
        pltpu.sync_copy(wt.at[pl.ds(k * N_PAD + base, BN)], w_v)

        def wslice(j, carry, first=(k == 0)):
            pltpu.sync_copy(inft.at[pl.ds(k * N_PAD + base + j * SL, SL)],
                            gidx_v)
            pltpu.async_copy(p_tab.at[gidx_v], rows_v, sem).wait()

            def wbody(g, c2):
                b = j * SL + g * L
                row = g * L + lanes
                pc = [plsc.load_gather(rows_v, [row, _col(c)])
                      for c in range(15)]
                vx = vx_v[pl.ds(b, L)]
                vy = vy_v[pl.ds(b, L)]
                vz = vz_v[pl.ds(b, L)]
                w = w_v[pl.ds(b, L)]
                dx = vx - pc[0]
                dy = vy - pc[1]
                dz = vz - pc[2]
                rx = pc[6] * dx + pc[7] * dy + pc[8] * dz
                ry = pc[9] * dx + pc[10] * dy + pc[11] * dz
                rz = pc[12] * dx + pc[13] * dy + pc[14] * dz
                ox = w * (rx + pc[0] + pc[3])
                oy = w * (ry + pc[1] + pc[4])
                oz = w * (rz + pc[2] + pc[5])
                flat = lanes3 + b * 3
                if first:
                    plsc.store_scatter(out_v, [flat], ox)
                    plsc.store_scatter(out_v, [flat + 1], oy)
                    plsc.store_scatter(out_v, [flat + 2], oz)
                else:
                    plsc.addupdate_scatter(out_v, [flat], ox)
                    plsc.addupdate_scatter(out_v, [flat + 1], oy)
                    plsc.addupdate_scatter(out_v, [flat + 2], oz)
                return c2

            lax.fori_loop(0, SL // L, wbody, 0)
            return carry

        lax.fori_loop(0, BN // SL, wslice, 0)
    pltpu.sync_copy(out_v, out_w.at[pl.ds(base * 3, BN * 3)])

    abase = wid * BA

    def asub(si, acc):
        p0 = abase + si * SUB
        pltpu.sync_copy(mrep.at[pl.ds(p0, SUB)], mi_v)
        m0 = pl.multiple_of(
            jnp.minimum((p0 // NEIGH) // 8 * 8, (M - 1) // 8 * 8), 8)
        pltpu.sync_copy(p_tab.at[pl.ds(m0, MROWS)], mrows_v)

        def aslice(j, acc2):
            pltpu.sync_copy(ornf.at[pl.ds(p0 + j * SL, SL)], gidx_v)
            pltpu.async_copy(p_tab.at[gidx_v], rows_v, sem).wait()

            def abody(g, acc3):
                aa, ss = acc3
                row = g * L + lanes
                ml = mi_v[pl.ds(j * SL + g * L, L)] - m0
                mc = [plsc.load_gather(mrows_v, [ml, _col(c)])
                      for c in range(15)]
                nc = [plsc.load_gather(rows_v, [row, _col(c)])
                      for c in range(15)]
                dx = mc[0] - nc[0]
                dy = mc[1] - nc[1]
                dz = mc[2] - nc[2]
                rx = mc[6] * dx + mc[7] * dy + mc[8] * dz
                ry = mc[9] * dx + mc[10] * dy + mc[11] * dz
                rz = mc[12] * dx + mc[13] * dy + mc[14] * dz
                ex = dx + (mc[3] - nc[3]) - rx
                ey = dy + (mc[4] - nc[4]) - ry
                ez = dz + (mc[5] - nc[5]) - rz
                aa = aa + ex * ex + ey * ey + ez * ez
                for c in range(9):
                    d = mc[6 + c] - nc[6 + c]
                    ss = ss + d * d
                return aa, ss

            return lax.fori_loop(0, SL // L, abody, acc2)

        return lax.fori_loop(0, SUB // SL, aslice, acc)

    acc_a, acc_s = lax.fori_loop(
        0, SCH, asub, (jnp.zeros((L,), jnp.float32),
                       jnp.zeros((L,), jnp.float32)))
    loss_v[pl.ds(0, L)] = acc_a
    loss_v[pl.ds(L, L)] = acc_s
    pltpu.sync_copy(loss_v, out_loss.at[pl.ds(wid * 2 * L, 2 * L)])


def kernel(vertices, opt_d_rotations, opt_d_translations, weights, nodes_idx,
           influence_nodes_idx, one_ring_neigh):
    rflat = jnp.pad(opt_d_rotations[0].reshape(M, 9).astype(jnp.float32),
                    ((0, M_PAD - M), (0, 0))).reshape(-1)
    tflat = jnp.pad(opt_d_translations[0].astype(jnp.float32),
                    ((0, M_PAD - M), (0, 0))).reshape(-1)
    nidx = jnp.pad(nodes_idx.astype(jnp.int32), (0, M_PAD - M))
    vt = jnp.pad(vertices.astype(jnp.float32),
                 ((0, N_PAD - N), (0, 0))).T.reshape(-1)
    inft = jnp.pad(influence_nodes_idx.astype(jnp.int32),
                   ((0, N_PAD - N), (0, 0))).T.reshape(-1)
    wt = jnp.pad(weights.astype(jnp.float32),
                 ((0, N_PAD - N), (0, 0))).T.reshape(-1)
    ornf = jnp.pad(one_ring_neigh.astype(jnp.int32).reshape(-1),
                   (0, MN_PAD - M * NEIGH), constant_values=M - 1)
    mrep = jnp.pad(jnp.repeat(jnp.arange(M, dtype=jnp.int32), NEIGH),
                   (0, MN_PAD - M * NEIGH), constant_values=M - 1)

    p_tab = _build_table(nidx, rflat, tflat, vt)
    out_w, out_loss = _warp_losses(p_tab, vt, inft, wt, ornf, mrep)

    warpped = out_w.reshape(N_PAD, 3)[:N][None]
    loss = out_loss.reshape(NW, 2 * L)
    arap = loss[:, :L].sum() / M
    sr = loss[:, L:].sum() / (M * NEIGH * 9)
    return (warpped, arap, sr)


# speedup vs baseline: 6.0977x; 1.0482x over previous
import functools

import jax
import jax.numpy as jnp
from jax import lax
from jax.experimental import pallas as pl
from jax.experimental.pallas import tpu as pltpu
from jax.experimental.pallas import tpu_sc as plsc

NC = 2
NS = 16
NW = NC * NS
L = 16
SL = 128
PW = 128

N = 50000
M = 25000
K = 3
NEIGH = 9

BN = 1664
N_PAD = NW * BN
N_V = 51200
BMW = 896
M_PAD = NW * BMW
SUB = 1024
SCH = 7
BA = SUB * SCH
MN_PAD = NW * BA
MROWS = 128

_mesh = plsc.VectorSubcoreMesh(core_axis_name="c", subcore_axis_name="s")
_params = pltpu.CompilerParams(needs_layout_passes=False)


def _col(c):
    return jnp.full((L,), c, jnp.int32)


@functools.partial(
    pl.kernel,
    out_type=jax.ShapeDtypeStruct((M_PAD, PW), jnp.float32),
    mesh=_mesh,
    compiler_params=_params,
    scratch_types=[
        pltpu.VMEM((N_V,), jnp.float32),
        pltpu.VMEM((BMW,), jnp.int32),
        pltpu.VMEM((BMW,), jnp.float32),
        pltpu.VMEM((BMW,), jnp.float32),
        pltpu.VMEM((BMW,), jnp.float32),
        pltpu.VMEM((BMW * 9,), jnp.float32),
        pltpu.VMEM((BMW * 3,), jnp.float32),
        pltpu.VMEM((SL, PW), jnp.float32),
    ],
)
def _build_table(nidx, rflat, tflat, vt, p_out,
                 vfull_v, nidx_v, nx_v, ny_v, nz_v, rr_v, tr_v, p_v):
    sid = lax.axis_index("s")
    cid = lax.axis_index("c")
    wid = sid * NC + cid
    lanes = lax.iota(jnp.int32, L)
    mb = wid * BMW

    pltpu.sync_copy(nidx.at[pl.ds(mb, BMW)], nidx_v)
    for c, nc_v in enumerate((nx_v, ny_v, nz_v)):
        pltpu.sync_copy(vt.at[pl.ds(c * N_PAD, N_V)], vfull_v)

        def gbody(g, carry, nc_v=nc_v):
            iv = nidx_v[pl.ds(g * L, L)]
            nc_v[pl.ds(g * L, L)] = plsc.load_gather(vfull_v, [iv])
            return carry

        lax.fori_loop(0, BMW // L, gbody, 0)
    pltpu.sync_copy(rflat.at[pl.ds(mb * 9, BMW * 9)], rr_v)
    pltpu.sync_copy(tflat.at[pl.ds(mb * 3, BMW * 3)], tr_v)

    def pslice(j, carry):
        def pbody(g, c2):
            row = g * L + lanes
            b = j * SL + g * L
            grow = b + lanes
            plsc.store_scatter(p_v, [row, _col(0)], nx_v[pl.ds(b, L)])
            plsc.store_scatter(p_v, [row, _col(1)], ny_v[pl.ds(b, L)])
            plsc.store_scatter(p_v, [row, _col(2)], nz_v[pl.ds(b, L)])
            f3 = grow * 3
            f9 = grow * 9
            for c in range(3):
                plsc.store_scatter(p_v, [row, _col(3 + c)],
                                   plsc.load_gather(tr_v, [f3 + c]))
            for c in range(9):
                plsc.store_scatter(p_v, [row, _col(6 + c)],
                                   plsc.load_gather(rr_v, [f9 + c]))
            return c2

        lax.fori_loop(0, SL // L, pbody, 0)
        pltpu.sync_copy(p_v, p_out.at[pl.ds(mb + j * SL, SL)])
        return carry

    lax.fori_loop(0, BMW // SL, pslice, 0)


@functools.partial(
    pl.kernel,
    out_type=(
        jax.ShapeDtypeStruct((N_PAD * 3,), jnp.float32),
        jax.ShapeDtypeStruct((NW * 2 * L,), jnp.float32),
    ),
    mesh=_mesh,
    compiler_params=_params,
    scratch_types=[
        pltpu.VMEM((SL,), jnp.int32),
        pltpu.VMEM((SL,), jnp.int32),
        pltpu.VMEM((BN,), jnp.float32),
        pltpu.VMEM((BN,), jnp.float32),
        pltpu.VMEM((BN,), jnp.float32),
        pltpu.VMEM((BN,), jnp.float32),
        pltpu.VMEM((SL, PW), jnp.float32),
        pltpu.VMEM((SL, PW), jnp.float32),
        pltpu.VMEM((MROWS, PW), jnp.float32),
        pltpu.VMEM((BN * 3,), jnp.float32),
        pltpu.VMEM((SUB,), jnp.int32),
        pltpu.VMEM((2 * L,), jnp.float32),
        pltpu.SemaphoreType.DMA,
        pltpu.SemaphoreType.DMA,
    ],
)
def _warp_losses(p_tab, vt, inft, wt, ornf, mrep, out_w, out_loss,
                 gidx_v, gidx_b, vx_v, vy_v, vz_v, w_v, rows_v, rows_b,
                 mrows_v, out_v, mi_v, loss_v, sem, sem_b):
    sid = lax.axis_index("s")
    cid = lax.axis_index("c")
    wid = sid * NC + cid
    lanes = lax.iota(jnp.int32, L)
    lanes3 = lanes * 3

    base = wid * BN
    pltpu.sync_copy(vt.at[pl.ds(base, BN)], vx_v)
    pltpu.sync_copy(vt.at[pl.ds(N_PAD + base, BN)], vy_v)
    pltpu.sync_copy(vt.at[pl.ds(2 * N_PAD + base, BN)], vz_v)
    for k in range(K):
        pltpu.sync_copy(wt.at[pl.ds(k * N_PAD + base, BN)], w_v)

        def wcompute(j, rows, first):
            def wbody(g, c2):
                b = j * SL + g * L
                row = g * L + lanes
                pc = [plsc.load_gather(rows, [row, _col(c)])
                      for c in range(15)]
                vx = vx_v[pl.ds(b, L)]
                vy = vy_v[pl.ds(b, L)]
                vz = vz_v[pl.ds(b, L)]
                w = w_v[pl.ds(b, L)]
                dx = vx - pc[0]
                dy = vy - pc[1]
                dz = vz - pc[2]
                rx = pc[6] * dx + pc[7] * dy + pc[8] * dz
                ry = pc[9] * dx + pc[10] * dy + pc[11] * dz
                rz = pc[12] * dx + pc[13] * dy + pc[14] * dz
                ox = w * (rx + pc[0] + pc[3])
                oy = w * (ry + pc[1] + pc[4])
                oz = w * (rz + pc[2] + pc[5])
                flat = lanes3 + b * 3
                if first:
                    plsc.store_scatter(out_v, [flat], ox)
                    plsc.store_scatter(out_v, [flat + 1], oy)
                    plsc.store_scatter(out_v, [flat + 2], oz)
                else:
                    plsc.addupdate_scatter(out_v, [flat], ox)
                    plsc.addupdate_scatter(out_v, [flat + 1], oy)
                    plsc.addupdate_scatter(out_v, [flat + 2], oz)
                return c2

            lax.fori_loop(0, SL // L, wbody, 0)

        def wpair(j2, carry, first=(k == 0)):
            j0 = j2 * 2
            ib = k * N_PAD + base
            pltpu.sync_copy(inft.at[pl.ds(ib + j0 * SL, SL)], gidx_v)
            cpa = pltpu.async_copy(p_tab.at[gidx_v], rows_v, sem)
            pltpu.sync_copy(inft.at[pl.ds(ib + (j0 + 1) * SL, SL)], gidx_b)
            cpb = pltpu.async_copy(p_tab.at[gidx_b], rows_b, sem_b)
            cpa.wait()
            wcompute(j0, rows_v, first)
            cpb.wait()
            wcompute(j0 + 1, rows_b, first)
            return carry

        lax.fori_loop(0, BN // SL // 2, wpair, 0)
        jt = BN // SL - 1
        pltpu.sync_copy(inft.at[pl.ds(k * N_PAD + base + jt * SL, SL)],
                        gidx_v)
        pltpu.async_copy(p_tab.at[gidx_v], rows_v, sem).wait()
        wcompute(jt, rows_v, (k == 0))
    pltpu.sync_copy(out_v, out_w.at[pl.ds(base * 3, BN * 3)])

    abase = wid * BA

    def asub(si, acc):
        p0 = abase + si * SUB
        pltpu.sync_copy(mrep.at[pl.ds(p0, SUB)], mi_v)
        m0 = pl.multiple_of(
            jnp.minimum((p0 // NEIGH) // 8 * 8, (M - 1) // 8 * 8), 8)
        pltpu.sync_copy(p_tab.at[pl.ds(m0, MROWS)], mrows_v)

        def acompute(j, rows, acc2):
            def abody(g, acc3):
                aa, ss = acc3
                row = g * L + lanes
                ml = mi_v[pl.ds(j * SL + g * L, L)] - m0
                mc = [plsc.load_gather(mrows_v, [ml, _col(c)])
                      for c in range(15)]
                nc = [plsc.load_gather(rows, [row, _col(c)])
                      for c in range(15)]
                dx = mc[0] - nc[0]
                dy = mc[1] - nc[1]
                dz = mc[2] - nc[2]
                rx = mc[6] * dx + mc[7] * dy + mc[8] * dz
                ry = mc[9] * dx + mc[10] * dy + mc[11] * dz
                rz = mc[12] * dx + mc[13] * dy + mc[14] * dz
                ex = dx + (mc[3] - nc[3]) - rx
                ey = dy + (mc[4] - nc[4]) - ry
                ez = dz + (mc[5] - nc[5]) - rz
                aa = aa + ex * ex + ey * ey + ez * ez
                for c in range(9):
                    d = mc[6 + c] - nc[6 + c]
                    ss = ss + d * d
                return aa, ss

            return lax.fori_loop(0, SL // L, abody, acc2)

        def apair(j2, acc2):
            j0 = j2 * 2
            pltpu.sync_copy(ornf.at[pl.ds(p0 + j0 * SL, SL)], gidx_v)
            cpa = pltpu.async_copy(p_tab.at[gidx_v], rows_v, sem)
            pltpu.sync_copy(ornf.at[pl.ds(p0 + (j0 + 1) * SL, SL)], gidx_b)
            cpb = pltpu.async_copy(p_tab.at[gidx_b], rows_b, sem_b)
            cpa.wait()
            acc2 = acompute(j0, rows_v, acc2)
            cpb.wait()
            return acompute(j0 + 1, rows_b, acc2)

        return lax.fori_loop(0, SUB // SL // 2, apair, acc)

    acc_a, acc_s = lax.fori_loop(
        0, SCH, asub, (jnp.zeros((L,), jnp.float32),
                       jnp.zeros((L,), jnp.float32)))
    loss_v[pl.ds(0, L)] = acc_a
    loss_v[pl.ds(L, L)] = acc_s
    pltpu.sync_copy(loss_v, out_loss.at[pl.ds(wid * 2 * L, 2 * L)])


def kernel(vertices, opt_d_rotations, opt_d_translations, weights, nodes_idx,
           influence_nodes_idx, one_ring_neigh):
    rflat = jnp.pad(opt_d_rotations[0].reshape(M, 9).astype(jnp.float32),
                    ((0, M_PAD - M), (0, 0))).reshape(-1)
    tflat = jnp.pad(opt_d_translations[0].astype(jnp.float32),
                    ((0, M_PAD - M), (0, 0))).reshape(-1)
    nidx = jnp.pad(nodes_idx.astype(jnp.int32), (0, M_PAD - M))
    vt = jnp.pad(vertices.astype(jnp.float32),
                 ((0, N_PAD - N), (0, 0))).T.reshape(-1)
    inft = jnp.pad(influence_nodes_idx.astype(jnp.int32),
                   ((0, N_PAD - N), (0, 0))).T.reshape(-1)
    wt = jnp.pad(weights.astype(jnp.float32),
                 ((0, N_PAD - N), (0, 0))).T.reshape(-1)
    ornf = jnp.pad(one_ring_neigh.astype(jnp.int32).reshape(-1),
                   (0, MN_PAD - M * NEIGH), constant_values=M - 1)
    mrep = jnp.pad(jnp.repeat(jnp.arange(M, dtype=jnp.int32), NEIGH),
                   (0, MN_PAD - M * NEIGH), constant_values=M - 1)

    p_tab = _build_table(nidx, rflat, tflat, vt)
    out_w, out_loss = _warp_losses(p_tab, vt, inft, wt, ornf, mrep)

    warpped = out_w.reshape(N_PAD, 3)[:N][None]
    loss = out_loss.reshape(NW, 2 * L)
    arap = loss[:, :L].sum() / M
    sr = loss[:, L:].sum() / (M * NEIGH * 9)
    return (warpped, arap, sr)
